# Initial kernel scaffold; baseline (speedup 1.0000x reference)
import jax
import jax.numpy as jnp
from jax.experimental import pallas as pl


def kernel(input, edge_index, adj_values, weight):
    raise NotImplementedError("write your pallas kernel here")


# speedup vs baseline: 3.0842x; 3.0842x over previous
import functools

import jax
import jax.numpy as jnp
from jax import lax
from jax.experimental import pallas as pl
from jax.experimental.pallas import tpu as pltpu
from jax.experimental.pallas import tpu_sc as plsc

_N = 10000
_E = 320000
_DIN = 128
_DOUT = 128
_DH = _DOUT // 2
_NS = 16
_EPT = _E // _NS
_C = 80
_G = _EPT // _C
_RPT = _N // _NS


def _matmul_body(x_ref, w_ref, o_ref):
    o_ref[...] = jnp.dot(x_ref[...], w_ref[...],
                         preferred_element_type=jnp.float32)


def _support_matmul(x, w):
    bm = 1000
    return pl.pallas_call(
        _matmul_body,
        grid=(_N // bm,),
        in_specs=[
            pl.BlockSpec((bm, _DIN), lambda i: (i, 0)),
            pl.BlockSpec((_DIN, _DOUT), lambda i: (0, 0)),
        ],
        out_specs=pl.BlockSpec((bm, _DOUT), lambda i: (i, 0)),
        out_shape=jax.ShapeDtypeStruct((_N, _DOUT), jnp.float32),
    )(x, w)


def _sc_body(table, src3, dst3, adj3, zeros, out,
             src_v, dst_v, adj_v, rows_v, acc, sem):
    c = lax.axis_index("c")
    s = lax.axis_index("s")

    pltpu.sync_copy(zeros, acc.at[pl.ds(s * _RPT, _RPT)])

    pltpu.sync_copy(src3.at[s], src_v)
    pltpu.sync_copy(dst3.at[s], dst_v)
    pltpu.sync_copy(adj3.at[s], adj_v)

    base = c * _N

    def rebase_row(g, carry):
        for k in range(_C // 16):
            sl = (g, pl.ds(k * 16, 16))
            src_v[sl] = src_v[sl] + base
        return carry

    lax.fori_loop(0, _G, rebase_row, 0)
    plsc.subcore_barrier()

    def chunk(g, carry):
        pltpu.async_copy(table.at[src_v.at[g]], rows_v, sem).wait()

        def edge_group(grp, inner):
            e0 = grp * 16
            av = adj_v[g, pl.ds(e0, 16)]
            for i in range(16):
                a = av[i]
                for j in range(_DH // 16):
                    sl = (e0 + i, pl.ds(j * 16, 16))
                    rows_v[sl] = rows_v[sl] * a
            return inner

        lax.fori_loop(0, _C // 16, edge_group, 0)
        pltpu.sync_copy(rows_v, acc.at[dst_v.at[g]], add=True)
        return carry

    lax.fori_loop(0, _G, chunk, 0)
    plsc.subcore_barrier()

    pltpu.sync_copy(acc.at[pl.ds(s * _RPT, _RPT)],
                    out.at[pl.ds(s * _RPT, _RPT), pl.ds(c * _DH, _DH)])


_sc_call = pl.kernel(
    _sc_body,
    mesh=plsc.VectorSubcoreMesh(core_axis_name="c", subcore_axis_name="s"),
    out_type=jax.ShapeDtypeStruct((_N, _DOUT), jnp.float32),
    scratch_types=[
        pltpu.VMEM((_G, _C), jnp.int32),
        pltpu.VMEM((_G, _C), jnp.int32),
        pltpu.VMEM((_G, _C), jnp.float32),
        pltpu.VMEM((_C, _DH), jnp.float32),
        pltpu.VMEM_SHARED((_N, _DH), jnp.float32),
        pltpu.SemaphoreType.DMA,
    ],
    compiler_params=pltpu.CompilerParams(use_tc_tiling_on_sc=False),
)


def kernel(input, edge_index, adj_values, weight):
    x = input.astype(jnp.float32)
    w = weight.astype(jnp.float32)
    support = _support_matmul(x, w)
    table = jnp.concatenate([support[:, :_DH], support[:, _DH:]], axis=0)
    src = edge_index[1].astype(jnp.int32).reshape(_NS, _G, _C)
    dst = edge_index[0].astype(jnp.int32).reshape(_NS, _G, _C)
    adj = adj_values.astype(jnp.float32).reshape(_NS, _G, _C)
    zeros = jnp.zeros((_RPT, _DH), jnp.float32)
    return _sc_call(table, src, dst, adj, zeros)


# speedup vs baseline: 4.3972x; 1.4257x over previous
import functools

import jax
import jax.numpy as jnp
from jax import lax
from jax.experimental import pallas as pl
from jax.experimental.pallas import tpu as pltpu
from jax.experimental.pallas import tpu_sc as plsc

_N = 10000
_E = 320000
_DIN = 128
_DOUT = 128
_DH = _DOUT // 2
_NS = 16
_EPT = _E // _NS
_C = 80
_G = _EPT // _C
_RPT = _N // _NS


def _matmul_body(x_ref, w_ref, o_ref):
    o_ref[...] = jnp.dot(x_ref[...], w_ref[...],
                         preferred_element_type=jnp.float32)


def _support_matmul(x, w):
    bm = 1000
    return pl.pallas_call(
        _matmul_body,
        grid=(_N // bm,),
        in_specs=[
            pl.BlockSpec((bm, _DIN), lambda i: (i, 0)),
            pl.BlockSpec((_DIN, _DOUT), lambda i: (0, 0)),
        ],
        out_specs=pl.BlockSpec((bm, _DOUT), lambda i: (i, 0)),
        out_shape=jax.ShapeDtypeStruct((_N, _DOUT), jnp.float32),
    )(x, w)


def _sc_body(table, src3, dst3, adj3, zeros, out,
             src_v, dst_v, adj_v, rows_v, acc, sem):
    c = lax.axis_index("c")
    s = lax.axis_index("s")

    pltpu.sync_copy(zeros, acc.at[pl.ds(s * _RPT, _RPT)])

    pltpu.sync_copy(src3.at[s], src_v)
    pltpu.sync_copy(dst3.at[s], dst_v)
    pltpu.sync_copy(adj3.at[s], adj_v)

    base = c * _N

    def rebase_row(g, carry):
        for k in range(_C // 16):
            sl = (g, pl.ds(k * 16, 16))
            src_v[sl] = src_v[sl] + base
        return carry

    lax.fori_loop(0, _G, rebase_row, 0)
    plsc.subcore_barrier()

    pltpu.async_copy(table.at[src_v.at[0]], rows_v.at[0], sem.at[0])

    def chunk(g, carry):
        b = lax.rem(g, 2)
        nb = 1 - b
        pltpu.make_async_copy(table.at[src_v.at[g]], rows_v.at[b],
                              sem.at[b]).wait()

        @pl.when(g + 1 < _G)
        def _prefetch():
            pltpu.async_copy(table.at[src_v.at[g + 1]], rows_v.at[nb],
                             sem.at[nb])

        def edge_group(grp, inner):
            e0 = grp * 16
            av = adj_v[g, pl.ds(e0, 16)]
            for i in range(16):
                a = av[i]
                for j in range(_DH // 16):
                    sl = (b, e0 + i, pl.ds(j * 16, 16))
                    rows_v[sl] = rows_v[sl] * a
            return inner

        lax.fori_loop(0, _C // 16, edge_group, 0)
        pltpu.sync_copy(rows_v.at[b], acc.at[dst_v.at[g]], add=True)
        return carry

    lax.fori_loop(0, _G, chunk, 0)
    plsc.subcore_barrier()

    pltpu.sync_copy(acc.at[pl.ds(s * _RPT, _RPT)],
                    out.at[pl.ds(s * _RPT, _RPT), pl.ds(c * _DH, _DH)])


_sc_call = pl.kernel(
    _sc_body,
    mesh=plsc.VectorSubcoreMesh(core_axis_name="c", subcore_axis_name="s"),
    out_type=jax.ShapeDtypeStruct((_N, _DOUT), jnp.float32),
    scratch_types=[
        pltpu.VMEM((_G, _C), jnp.int32),
        pltpu.VMEM((_G, _C), jnp.int32),
        pltpu.VMEM((_G, _C), jnp.float32),
        pltpu.VMEM((2, _C, _DH), jnp.float32),
        pltpu.VMEM_SHARED((_N, _DH), jnp.float32),
        pltpu.SemaphoreType.DMA((2,)),
    ],
    compiler_params=pltpu.CompilerParams(use_tc_tiling_on_sc=False),
)


def kernel(input, edge_index, adj_values, weight):
    x = input.astype(jnp.float32)
    w = weight.astype(jnp.float32)
    support = _support_matmul(x, w)
    table = jnp.concatenate([support[:, :_DH], support[:, _DH:]], axis=0)
    src = edge_index[1].astype(jnp.int32).reshape(_NS, _G, _C)
    dst = edge_index[0].astype(jnp.int32).reshape(_NS, _G, _C)
    adj = adj_values.astype(jnp.float32).reshape(_NS, _G, _C)
    zeros = jnp.zeros((_RPT, _DH), jnp.float32)
    return _sc_call(table, src, dst, adj, zeros)


# speedup vs baseline: 5.0347x; 1.1450x over previous
import functools

import jax
import jax.numpy as jnp
from jax import lax
from jax.experimental import pallas as pl
from jax.experimental.pallas import tpu as pltpu
from jax.experimental.pallas import tpu_sc as plsc

_N = 10000
_E = 320000
_DIN = 128
_DOUT = 128
_DH = _DOUT // 2
_NS = 16
_EPT = _E // _NS
_C = 80
_G = _EPT // _C
_RPT = _N // _NS
_NB = 4


def _matmul_body(x_ref, w_ref, o_ref):
    o_ref[...] = jnp.dot(x_ref[...], w_ref[...],
                         preferred_element_type=jnp.float32)


def _support_matmul(x, w):
    bm = 1000
    return pl.pallas_call(
        _matmul_body,
        grid=(_N // bm,),
        in_specs=[
            pl.BlockSpec((bm, _DIN), lambda i: (i, 0)),
            pl.BlockSpec((_DIN, _DOUT), lambda i: (0, 0)),
        ],
        out_specs=pl.BlockSpec((bm, _DOUT), lambda i: (i, 0)),
        out_shape=jax.ShapeDtypeStruct((_N, _DOUT), jnp.float32),
    )(x, w)


def _sc_body(table, src3, dst3, adj3, zeros, out,
             src_v, dst_v, adj_v, rows_v, acc, gsem, ssem):
    c = lax.axis_index("c")
    s = lax.axis_index("s")

    pltpu.sync_copy(zeros, acc.at[pl.ds(s * _RPT, _RPT)])

    pltpu.sync_copy(src3.at[s], src_v)
    pltpu.sync_copy(dst3.at[s], dst_v)
    pltpu.sync_copy(adj3.at[s], adj_v)

    base = c * _N

    def rebase_row(g, carry):
        for k in range(_C // 16):
            sl = (g, pl.ds(k * 16, 16))
            src_v[sl] = src_v[sl] + base
        return carry

    lax.fori_loop(0, _G, rebase_row, 0)
    plsc.subcore_barrier()

    for k in range(_NB):
        pltpu.async_copy(table.at[src_v.at[k]], rows_v.at[k], gsem.at[k])

    def chunk(g, carry):
        b = lax.rem(g, _NB)
        pltpu.make_async_copy(table.at[src_v.at[g]], rows_v.at[b],
                              gsem.at[b]).wait()

        def edge_group(grp, inner):
            e0 = grp * 16
            av = adj_v[g, pl.ds(e0, 16)]
            for i in range(16):
                a = av[i]
                for j in range(_DH // 16):
                    sl = (b, e0 + i, pl.ds(j * 16, 16))
                    rows_v[sl] = rows_v[sl] * a
            return inner

        lax.fori_loop(0, _C // 16, edge_group, 0)
        pltpu.async_copy(rows_v.at[b], acc.at[dst_v.at[g]], ssem.at[b],
                         add=True)

        @pl.when((g >= 1) & (g + (_NB - 1) < _G))
        def _prefetch():
            pb = lax.rem(g - 1, _NB)
            pltpu.make_async_copy(rows_v.at[pb], acc.at[dst_v.at[g - 1]],
                                  ssem.at[pb]).wait()
            pltpu.async_copy(table.at[src_v.at[g + (_NB - 1)]],
                             rows_v.at[pb], gsem.at[pb])

        return carry

    lax.fori_loop(0, _G, chunk, 0)

    for k in range(_G - _NB, _G):
        b = k % _NB
        pltpu.make_async_copy(rows_v.at[b], acc.at[dst_v.at[k]],
                              ssem.at[b]).wait()
    plsc.subcore_barrier()

    pltpu.sync_copy(acc.at[pl.ds(s * _RPT, _RPT)],
                    out.at[pl.ds(s * _RPT, _RPT), pl.ds(c * _DH, _DH)])


_sc_call = pl.kernel(
    _sc_body,
    mesh=plsc.VectorSubcoreMesh(core_axis_name="c", subcore_axis_name="s"),
    out_type=jax.ShapeDtypeStruct((_N, _DOUT), jnp.float32),
    scratch_types=[
        pltpu.VMEM((_G, _C), jnp.int32),
        pltpu.VMEM((_G, _C), jnp.int32),
        pltpu.VMEM((_G, _C), jnp.float32),
        pltpu.VMEM((_NB, _C, _DH), jnp.float32),
        pltpu.VMEM_SHARED((_N, _DH), jnp.float32),
        pltpu.SemaphoreType.DMA((_NB,)),
        pltpu.SemaphoreType.DMA((_NB,)),
    ],
    compiler_params=pltpu.CompilerParams(use_tc_tiling_on_sc=False),
)


def kernel(input, edge_index, adj_values, weight):
    x = input.astype(jnp.float32)
    w = weight.astype(jnp.float32)
    support = _support_matmul(x, w)
    table = jnp.concatenate([support[:, :_DH], support[:, _DH:]], axis=0)
    src = edge_index[1].astype(jnp.int32).reshape(_NS, _G, _C)
    dst = edge_index[0].astype(jnp.int32).reshape(_NS, _G, _C)
    adj = adj_values.astype(jnp.float32).reshape(_NS, _G, _C)
    zeros = jnp.zeros((_RPT, _DH), jnp.float32)
    return _sc_call(table, src, dst, adj, zeros)
